# Initial kernel scaffold; baseline (speedup 1.0000x reference)
import jax
import jax.numpy as jnp
from jax.experimental import pallas as pl


def kernel(x, edge_index, species, W1, att_src1, att_dst1, b1, W2, att_src2, att_dst2, b2, species_table, Wa, ba, Wb, bb, Wc, bc):
    raise NotImplementedError("write your pallas kernel here")


# speedup vs baseline: 23.8266x; 23.8266x over previous
import functools

import jax
import jax.numpy as jnp
from jax import lax
from jax.experimental import pallas as pl
from jax.experimental.pallas import tpu as pltpu
from jax.experimental.pallas import tpu_sc as plsc

N = 10000
E = 160000
NTILES = 16
F32 = jnp.float32
I32 = jnp.int32


_BN1 = 2000


def _tc1_body(x_ref, xt_ref, w1_ref, ast_ref, adt_ref, h1_ref, asp_ref, adp_ref):
    xb = x_ref[...]
    h1_ref[0] = jnp.dot(xb, w1_ref[0], precision=lax.Precision.HIGHEST,
                        preferred_element_type=F32)

    @pl.when((pl.program_id(1) == 0) & (pl.program_id(0) % 2 == 0))
    def _():
        xt = xt_ref[...]
        asp_ref[0] = jnp.dot(ast_ref[0], xt, precision=lax.Precision.HIGHEST,
                             preferred_element_type=F32)
        adp_ref[0] = jnp.dot(adt_ref[0], xt, precision=lax.Precision.HIGHEST,
                             preferred_element_type=F32)


def _tc1(x, xt, W1, AsT, AdT):
    nb = N // _BN1
    return pl.pallas_call(
        _tc1_body,
        grid=(4, nb),
        in_specs=[
            pl.BlockSpec((_BN1, 15), lambda h, i: (i, 0)),
            pl.BlockSpec((15, N), lambda h, i: (0, 0)),
            pl.BlockSpec((1, 15, 64), lambda h, i: (h, 0, 0)),
            pl.BlockSpec((1, 2, 15), lambda h, i: (h // 2, 0, 0)),
            pl.BlockSpec((1, 2, 15), lambda h, i: (h // 2, 0, 0)),
        ],
        out_specs=[
            pl.BlockSpec((1, _BN1, 64), lambda h, i: (h, i, 0)),
            pl.BlockSpec((1, 2, N), lambda h, i: (h // 2, 0, 0)),
            pl.BlockSpec((1, 2, N), lambda h, i: (h // 2, 0, 0)),
        ],
        out_shape=[
            jax.ShapeDtypeStruct((4, N, 64), F32),
            jax.ShapeDtypeStruct((2, 2, N), F32),
            jax.ShapeDtypeStruct((2, 2, N), F32),
        ],
    )(x, xt, W1, AsT, AdT)


_KA = 200
_EPT_A = E // NTILES
_NCH_A = _EPT_A // _KA
_ROWS_T = N // NTILES


def _full16(v):
    return jnp.full((16,), v, I32)


def _lrelu(v):
    return jnp.where(v > 0, v, 0.2 * v)


def _zero_rows(ref, nrows, ncols):
    z = jnp.zeros((16,), F32)

    def body(r, _):
        for k in range(ncols // 16):
            ref[r, pl.ds(16 * k, 16)] = z
        return 0

    lax.fori_loop(0, nrows, body, 0)


def _bcast_max(v, tmp16):
    iota = lax.iota(I32, 16)
    for sh in (1, 2, 4, 8):
        tmp16[...] = v
        v = jnp.maximum(v, plsc.load_gather(tmp16, [jnp.bitwise_xor(iota, sh)]))
    return v


def _table_max(tab_ref, row, n, tmp16):
    def body(i, acc):
        return jnp.maximum(acc, tab_ref[row, pl.ds(16 * i, 16)])

    acc = lax.fori_loop(0, n // 16, body, jnp.full((16,), -jnp.inf, F32))
    return _bcast_max(acc, tmp16)


def _sca_body(h1_hbm, asp_hbm, adp_hbm, src_hbm, dst_hbm, srcoff_hbm,
              m_hbm, den_hbm,
              as_t, ad_t, src_c, dst_c, srcoff_c, buf, exb, tmp16, acc_sp,
              den_sp, sem):
    c = lax.axis_index("c")
    s = lax.axis_index("s")

    pltpu.sync_copy(asp_hbm.at[c], as_t)
    pltpu.sync_copy(adp_hbm.at[c], ad_t)

    ebase = s * _EPT_A
    base_r = s * 1000
    iota = lax.iota(I32, 16)

    for j in range(2):
        cj = _lrelu(_table_max(as_t, j, N, tmp16)
                    + _table_max(ad_t, j, N, tmp16))
        rowoff = (2 * c + j) * N

        _zero_rows(buf, _KA, 64)
        _zero_rows(exb, _KA, 8)

        @pl.when(s < 10)
        def _():
            for o in range(0, 1000, _KA):
                pltpu.sync_copy(buf.at[pl.ds(0, _KA)],
                                acc_sp.at[pl.ds(base_r + o, _KA)])
                pltpu.sync_copy(exb.at[pl.ds(0, _KA)],
                                den_sp.at[pl.ds(base_r + o, _KA)])

        plsc.subcore_barrier()

        def chunk(g, _):
            off = ebase + g * _KA
            pltpu.sync_copy(src_hbm.at[pl.ds(off, _KA)], src_c)
            pltpu.sync_copy(dst_hbm.at[pl.ds(off, _KA)], dst_c)
            pltpu.sync_copy(srcoff_hbm.at[pl.ds((2 * c + j) * E + off, _KA)],
                            srcoff_c)

            pltpu.async_copy(h1_hbm.at[srcoff_c], buf, sem).wait()

            def group(i, _):
                srcv = src_c[pl.ds(16 * i, 16)]
                dstv = dst_c[pl.ds(16 * i, 16)]
                rows = iota + 16 * i
                a0 = plsc.load_gather(as_t, [_full16(j), srcv])
                d0 = plsc.load_gather(ad_t, [_full16(j), dstv])
                ex = jnp.exp(_lrelu(a0 + d0) - cj)
                plsc.store_scatter(exb, [rows, _full16(0)], ex)
                for e in range(16):
                    be = 16 * i + e
                    b0 = jnp.full((16,), ex[e])
                    for q in range(4):
                        buf[be, pl.ds(16 * q, 16)] = (
                            buf[be, pl.ds(16 * q, 16)] * b0)
                return 0

            lax.fori_loop(0, _KA // 16, group, 0)

            pltpu.sync_copy(buf, acc_sp.at[dst_c], add=True)
            pltpu.sync_copy(exb, den_sp.at[dst_c], add=True)
            return 0

        lax.fori_loop(0, _NCH_A, chunk, 0)
        plsc.subcore_barrier()

        out_r = rowoff + base_r

        @pl.when(s < 10)
        def _():
            for o in range(0, 1000, _KA):
                pltpu.sync_copy(acc_sp.at[pl.ds(base_r + o, _KA)],
                                buf.at[pl.ds(0, _KA)])
                pltpu.sync_copy(buf.at[pl.ds(0, _KA)],
                                m_hbm.at[pl.ds(out_r + o, _KA)])
                pltpu.sync_copy(den_sp.at[pl.ds(base_r + o, _KA)],
                                exb.at[pl.ds(0, _KA)])
                pltpu.sync_copy(exb.at[pl.ds(0, _KA)],
                                den_hbm.at[pl.ds(out_r + o, _KA)])

        plsc.subcore_barrier()


def _sca(h1flat, asp, adp, src, dst, srcoff4):
    mesh = plsc.VectorSubcoreMesh(core_axis_name="c", subcore_axis_name="s", num_cores=2, num_subcores=16)
    f = pl.kernel(
        _sca_body,
        out_type=[
            jax.ShapeDtypeStruct((4 * N, 64), F32),
            jax.ShapeDtypeStruct((4 * N, 8), F32),
        ],
        mesh=mesh,
        compiler_params=pltpu.CompilerParams(needs_layout_passes=False,
                                             use_tc_tiling_on_sc=False),
        scratch_types=[
            pltpu.VMEM((2, N), F32),
            pltpu.VMEM((2, N), F32),
            pltpu.VMEM((_KA,), I32),
            pltpu.VMEM((_KA,), I32),
            pltpu.VMEM((_KA,), I32),
            pltpu.VMEM((_KA, 64), F32),
            pltpu.VMEM((_KA, 8), F32),
            pltpu.VMEM((16,), F32),
            pltpu.VMEM_SHARED((N, 64), F32),
            pltpu.VMEM_SHARED((N, 8), F32),
            pltpu.SemaphoreType.DMA,
        ],
    )
    return f(h1flat, asp, adp, src, dst, srcoff4)


_BN2 = 400


def _tc2_body(m_ref, den_ref, b1_ref, w2_ref, att2_ref, h2_ref, a2_ref):
    b1 = b1_ref[...]
    feats = []
    for h in range(4):
        mh = m_ref[h]
        dh = den_ref[h, :, 0:1]
        denr = jnp.repeat(dh, 64, axis=1)
        o = jnp.where(denr > 0, mh / denr, 0.0) + b1[:, 64 * h:64 * (h + 1)]
        feats.append(jnp.where(o > 0, o, jnp.exp(o) - 1.0))
    hfeat = jnp.concatenate(feats, axis=1)
    h2_ref[...] = jnp.dot(hfeat, w2_ref[...], precision=lax.Precision.HIGHEST,
                          preferred_element_type=F32)
    a2w = jnp.dot(w2_ref[...], att2_ref[...], precision=lax.Precision.HIGHEST,
                  preferred_element_type=F32)
    a2_ref[...] = jnp.dot(hfeat, a2w, precision=lax.Precision.HIGHEST,
                          preferred_element_type=F32)


def _tc2(M, den, b1, W2, attcat2):
    nb = N // _BN2
    return pl.pallas_call(
        _tc2_body,
        grid=(nb,),
        in_specs=[
            pl.BlockSpec((4, _BN2, 64), lambda i: (0, i, 0)),
            pl.BlockSpec((4, _BN2, 8), lambda i: (0, i, 0)),
            pl.BlockSpec((1, 256), lambda i: (0, 0)),
            pl.BlockSpec((256, 64), lambda i: (0, 0)),
            pl.BlockSpec((64, 2), lambda i: (0, 0)),
        ],
        out_specs=[
            pl.BlockSpec((_BN2, 64), lambda i: (i, 0)),
            pl.BlockSpec((_BN2, 2), lambda i: (i, 0)),
        ],
        out_shape=[
            jax.ShapeDtypeStruct((N, 64), F32),
            jax.ShapeDtypeStruct((N, 2), F32),
        ],
    )(M, den, b1, W2, attcat2)


_KB = 1000
_EPT_B = E // 32
_NCH_B = _EPT_B // _KB


def _a2_max(a2_t, col, tmp16):
    def body(i, acc):
        rows = lax.iota(I32, 16) + 16 * i
        return jnp.maximum(acc, plsc.load_gather(a2_t, [rows, _full16(col)]))

    acc = lax.fori_loop(0, N // 16, body, jnp.full((16,), -jnp.inf, F32))
    return _bcast_max(acc, tmp16)


def _zero_flat(ref, n):
    z = jnp.zeros((16,), F32)

    def body(i, _):
        ref[pl.ds(16 * i, 16)] = z
        return 0

    lax.fori_loop(0, n // 16, body, 0)


def _scb_body(a2_hbm, src_hbm, dst_hbm, den2_hbm,
              a2_t, src_c, dst_c, exb, tmp16, den_sp):
    c = lax.axis_index("c")
    s = lax.axis_index("s")
    pltpu.sync_copy(a2_hbm, a2_t)
    c2 = _lrelu(_a2_max(a2_t, 0, tmp16) + _a2_max(a2_t, 1, tmp16))

    _zero_flat(exb, _KB)

    @pl.when(s < 10)
    def _():
        pltpu.sync_copy(exb, den_sp.at[pl.ds(s * 1000, 1000)])

    plsc.subcore_barrier()

    wid = s * 2 + c
    ebase = wid * _EPT_B

    def chunk(g, _):
        off = ebase + g * _KB
        pltpu.sync_copy(src_hbm.at[pl.ds(off, _KB)], src_c)
        pltpu.sync_copy(dst_hbm.at[pl.ds(off, _KB)], dst_c)

        def group(i, _):
            srcv = src_c[pl.ds(16 * i, 16)]
            dstv = dst_c[pl.ds(16 * i, 16)]
            av = plsc.load_gather(a2_t, [srcv, _full16(0)])
            dv = plsc.load_gather(a2_t, [dstv, _full16(1)])
            exb[pl.ds(16 * i, 16)] = jnp.exp(_lrelu(av + dv) - c2)
            return 0

        lax.fori_loop(0, _KB // 16, group, 0)
        pltpu.sync_copy(exb, den_sp.at[dst_c], add=True)
        return 0

    lax.fori_loop(0, _NCH_B, chunk, 0)
    plsc.subcore_barrier()

    @pl.when(s < 10)
    def _():
        pltpu.sync_copy(den_sp.at[pl.ds(s * 1000, 1000)], exb)
        pltpu.sync_copy(exb, den2_hbm.at[pl.ds(c * N + s * 1000, 1000)])


def _scb(a2, src, dst):
    mesh = plsc.VectorSubcoreMesh(core_axis_name="c", subcore_axis_name="s", num_cores=2, num_subcores=16)
    f = pl.kernel(
        _scb_body,
        out_type=jax.ShapeDtypeStruct((2 * N,), F32),
        mesh=mesh,
        compiler_params=pltpu.CompilerParams(needs_layout_passes=False,
                                             use_tc_tiling_on_sc=False),
        scratch_types=[
            pltpu.VMEM((N, 2), F32),
            pltpu.VMEM((_KB,), I32),
            pltpu.VMEM((_KB,), I32),
            pltpu.VMEM((_KB,), F32),
            pltpu.VMEM((16,), F32),
            pltpu.VMEM_SHARED((N,), F32),
        ],
    )
    return f(a2, src, dst)


def _scc_body(a2_hbm, den2_hbm, src_hbm, dst_hbm, w_hbm,
              a2_t, den2_t, dpart, src_c, dst_c, cb, tmp16, w_sp):
    c = lax.axis_index("c")
    s = lax.axis_index("s")
    pltpu.sync_copy(a2_hbm, a2_t)
    c2 = _lrelu(_a2_max(a2_t, 0, tmp16) + _a2_max(a2_t, 1, tmp16))

    pltpu.sync_copy(den2_hbm.at[pl.ds(0, N)], den2_t)
    pltpu.sync_copy(den2_hbm.at[pl.ds(N, N)], dpart)

    def addp(i, _):
        sl = pl.ds(16 * i, 16)
        den2_t[sl] = den2_t[sl] + dpart[sl]
        return 0

    lax.fori_loop(0, N // 16, addp, 0)

    _zero_flat(cb, _KB)

    @pl.when(s < 10)
    def _():
        pltpu.sync_copy(cb, w_sp.at[pl.ds(s * 1000, 1000)])

    plsc.subcore_barrier()

    wid = s * 2 + c
    ebase = wid * _EPT_B

    def chunk(g, _):
        off = ebase + g * _KB
        pltpu.sync_copy(src_hbm.at[pl.ds(off, _KB)], src_c)
        pltpu.sync_copy(dst_hbm.at[pl.ds(off, _KB)], dst_c)

        def group(i, _):
            srcv = src_c[pl.ds(16 * i, 16)]
            dstv = dst_c[pl.ds(16 * i, 16)]
            av = plsc.load_gather(a2_t, [srcv, _full16(0)])
            dv = plsc.load_gather(a2_t, [dstv, _full16(1)])
            ex = jnp.exp(_lrelu(av + dv) - c2)
            den = plsc.load_gather(den2_t, [dstv])
            cb[pl.ds(16 * i, 16)] = ex / den
            return 0

        lax.fori_loop(0, _KB // 16, group, 0)
        pltpu.sync_copy(cb, w_sp.at[src_c], add=True)
        return 0

    lax.fori_loop(0, _NCH_B, chunk, 0)
    plsc.subcore_barrier()

    @pl.when(s < 10)
    def _():
        pltpu.sync_copy(w_sp.at[pl.ds(s * 1000, 1000)], cb)
        pltpu.sync_copy(cb, w_hbm.at[pl.ds(c * N + s * 1000, 1000)])


def _scc(a2, den2, src, dst):
    mesh = plsc.VectorSubcoreMesh(core_axis_name="c", subcore_axis_name="s", num_cores=2, num_subcores=16)
    f = pl.kernel(
        _scc_body,
        out_type=jax.ShapeDtypeStruct((2 * N,), F32),
        mesh=mesh,
        compiler_params=pltpu.CompilerParams(needs_layout_passes=False,
                                             use_tc_tiling_on_sc=False),
        scratch_types=[
            pltpu.VMEM((N, 2), F32),
            pltpu.VMEM((N,), F32),
            pltpu.VMEM((N,), F32),
            pltpu.VMEM((_KB,), I32),
            pltpu.VMEM((_KB,), I32),
            pltpu.VMEM((_KB,), F32),
            pltpu.VMEM((16,), F32),
            pltpu.VMEM_SHARED((N,), F32),
        ],
    )
    return f(a2, den2, src, dst)


def _tc3_body(h2_ref, w_ref, b2_ref, sp_ref, wa_ref, ba_ref, wb_ref, bb_ref,
              wc_ref, bc_ref, o_ref):
    w = (w_ref[0:1, :] + w_ref[1:2, :])
    ge = jnp.dot(w, h2_ref[...], precision=lax.Precision.HIGHEST,
                 preferred_element_type=F32) / float(N) + b2_ref[...]
    cc = jnp.concatenate([ge, sp_ref[...]], axis=1)
    m = jnp.dot(cc, wa_ref[...], precision=lax.Precision.HIGHEST,
                preferred_element_type=F32) + ba_ref[...]
    m = jnp.maximum(m, 0.0)
    m = jnp.dot(m, wb_ref[...], precision=lax.Precision.HIGHEST,
                preferred_element_type=F32) + bb_ref[...]
    m = jnp.maximum(m, 0.0)
    o = jnp.dot(m, wc_ref[...], precision=lax.Precision.HIGHEST,
                preferred_element_type=F32) + bc_ref[...]
    o_ref[...] = 1.0 / (1.0 + jnp.exp(-o))


def _tc3(h2, wpart, b2, sp, Wa, ba, Wb, bb, Wc, bc):
    return pl.pallas_call(
        _tc3_body,
        out_shape=jax.ShapeDtypeStruct((1, 1), F32),
    )(h2, wpart, b2, sp, Wa, ba, Wb, bb, Wc, bc)


def kernel(x, edge_index, species, W1, att_src1, att_dst1, b1, W2, att_src2,
           att_dst2, b2, species_table, Wa, ba, Wb, bb, Wc, bc):
    src = edge_index[0]
    dst = edge_index[1]

    As1 = jnp.einsum("khc,hc->kh", W1.reshape(15, 4, 64), att_src1)
    Ad1 = jnp.einsum("khc,hc->kh", W1.reshape(15, 4, 64), att_dst1)
    AsT = As1.T.reshape(2, 2, 15)
    AdT = Ad1.T.reshape(2, 2, 15)

    W1h = W1.reshape(15, 4, 64).transpose(1, 0, 2)
    h1p, asp, adp = _tc1(x, x.T, W1h, AsT, AdT)

    srcoff4 = (src[None, :]
               + (jnp.arange(4, dtype=I32) * N)[:, None]).reshape(4 * E)
    Mflat, denflat = _sca(h1p.reshape(4 * N, 64), asp, adp, src, dst, srcoff4)
    M = Mflat.reshape(4, N, 64)
    den = denflat.reshape(4, N, 8)

    attcat2 = jnp.stack([att_src2[0], att_dst2[0]], axis=1)
    h2, a2 = _tc2(M, den, b1.reshape(1, 256), W2, attcat2)

    den2 = _scb(a2, src, dst)
    wflat = _scc(a2, den2, src, dst)

    sp = species_table[species][None, :]
    out = _tc3(h2, wflat.reshape(2, N), b2.reshape(1, 64), sp,
               Wa, ba.reshape(1, 128), Wb, bb.reshape(1, 64), Wc,
               bc.reshape(1, 1))
    return out.reshape(1)


# speedup vs baseline: 26.0820x; 1.0947x over previous
import functools

import jax
import jax.numpy as jnp
from jax import lax
from jax.experimental import pallas as pl
from jax.experimental.pallas import tpu as pltpu
from jax.experimental.pallas import tpu_sc as plsc

N = 10000
E = 160000
NTILES = 16
F32 = jnp.float32
I32 = jnp.int32


_BN1 = 2000


def _tc1_body(x_ref, xt_ref, w1_ref, ast_ref, adt_ref, h1_ref, asp_ref, adp_ref):
    xb = x_ref[...]
    h1_ref[0] = jnp.dot(xb, w1_ref[0], precision=lax.Precision.HIGHEST,
                        preferred_element_type=F32)

    @pl.when((pl.program_id(1) == 0) & (pl.program_id(0) % 2 == 0))
    def _():
        xt = xt_ref[...]
        asp_ref[0] = jnp.dot(ast_ref[0], xt, precision=lax.Precision.HIGHEST,
                             preferred_element_type=F32)
        adp_ref[0] = jnp.dot(adt_ref[0], xt, precision=lax.Precision.HIGHEST,
                             preferred_element_type=F32)


def _tc1(x, xt, W1, AsT, AdT):
    nb = N // _BN1
    return pl.pallas_call(
        _tc1_body,
        grid=(4, nb),
        in_specs=[
            pl.BlockSpec((_BN1, 15), lambda h, i: (i, 0)),
            pl.BlockSpec((15, N), lambda h, i: (0, 0)),
            pl.BlockSpec((1, 15, 64), lambda h, i: (h, 0, 0)),
            pl.BlockSpec((1, 2, 15), lambda h, i: (h // 2, 0, 0)),
            pl.BlockSpec((1, 2, 15), lambda h, i: (h // 2, 0, 0)),
        ],
        out_specs=[
            pl.BlockSpec((1, _BN1, 64), lambda h, i: (h, i, 0)),
            pl.BlockSpec((1, 2, N), lambda h, i: (h // 2, 0, 0)),
            pl.BlockSpec((1, 2, N), lambda h, i: (h // 2, 0, 0)),
        ],
        out_shape=[
            jax.ShapeDtypeStruct((4, N, 64), F32),
            jax.ShapeDtypeStruct((2, 2, N), F32),
            jax.ShapeDtypeStruct((2, 2, N), F32),
        ],
    )(x, xt, W1, AsT, AdT)


_KA = 400
_EPT_A = E // NTILES
_NCH_A = _EPT_A // _KA
_ROWS_T = N // NTILES


def _full16(v):
    return jnp.full((16,), v, I32)


def _lrelu(v):
    return jnp.where(v > 0, v, 0.2 * v)


def _zero_rows(ref, nrows, ncols):
    z = jnp.zeros((16,), F32)

    def body(r, _):
        for k in range(ncols // 16):
            ref[r, pl.ds(16 * k, 16)] = z
        return 0

    lax.fori_loop(0, nrows, body, 0)


def _bcast_max(v, tmp16):
    iota = lax.iota(I32, 16)
    for sh in (1, 2, 4, 8):
        tmp16[...] = v
        v = jnp.maximum(v, plsc.load_gather(tmp16, [jnp.bitwise_xor(iota, sh)]))
    return v


def _table_max(tab_ref, row, n, tmp16):
    def body(i, acc):
        return jnp.maximum(acc, tab_ref[row, pl.ds(16 * i, 16)])

    acc = lax.fori_loop(0, n // 16, body, jnp.full((16,), -jnp.inf, F32))
    return _bcast_max(acc, tmp16)


def _sca_body(h1_hbm, asp_hbm, adp_hbm, src_hbm, dst_hbm, srcoff_hbm,
              m_hbm, den_hbm,
              as_t, ad_t, src_c, dst_c, srcoff_c, buf, exb, tmp16, acc_sp,
              den_sp, sem):
    c = lax.axis_index("c")
    s = lax.axis_index("s")

    pltpu.sync_copy(asp_hbm.at[c], as_t)
    pltpu.sync_copy(adp_hbm.at[c], ad_t)

    ebase = s * _EPT_A
    base_r = s * 1000
    iota = lax.iota(I32, 16)

    for j in range(2):
        cj = _lrelu(_table_max(as_t, j, N, tmp16)
                    + _table_max(ad_t, j, N, tmp16))
        rowoff = (2 * c + j) * N

        _zero_rows(buf, _KA, 64)
        _zero_rows(exb, _KA, 8)

        @pl.when(s < 10)
        def _():
            for (o, nr) in ((0, 400), (400, 400), (800, 200)):
                pltpu.sync_copy(buf.at[pl.ds(0, nr)],
                                acc_sp.at[pl.ds(base_r + o, nr)])
                pltpu.sync_copy(exb.at[pl.ds(0, nr)],
                                den_sp.at[pl.ds(base_r + o, nr)])

        plsc.subcore_barrier()

        def chunk(g, _):
            off = ebase + g * _KA
            pltpu.sync_copy(src_hbm.at[pl.ds(off, _KA)], src_c)
            pltpu.sync_copy(dst_hbm.at[pl.ds(off, _KA)], dst_c)
            pltpu.sync_copy(srcoff_hbm.at[pl.ds((2 * c + j) * E + off, _KA)],
                            srcoff_c)

            pltpu.async_copy(h1_hbm.at[srcoff_c], buf, sem).wait()

            def group(i, _):
                srcv = src_c[pl.ds(16 * i, 16)]
                dstv = dst_c[pl.ds(16 * i, 16)]
                rows = iota + 16 * i
                a0 = plsc.load_gather(as_t, [_full16(j), srcv])
                d0 = plsc.load_gather(ad_t, [_full16(j), dstv])
                ex = jnp.exp(_lrelu(a0 + d0) - cj)
                plsc.store_scatter(exb, [rows, _full16(0)], ex)
                for e in range(16):
                    be = 16 * i + e
                    b0 = jnp.full((16,), ex[e])
                    for q in range(4):
                        buf[be, pl.ds(16 * q, 16)] = (
                            buf[be, pl.ds(16 * q, 16)] * b0)
                return 0

            lax.fori_loop(0, _KA // 16, group, 0)

            pltpu.sync_copy(buf, acc_sp.at[dst_c], add=True)
            pltpu.sync_copy(exb, den_sp.at[dst_c], add=True)
            return 0

        lax.fori_loop(0, _NCH_A, chunk, 0)
        plsc.subcore_barrier()

        out_r = rowoff + base_r

        @pl.when(s < 10)
        def _():
            for (o, nr) in ((0, 400), (400, 400), (800, 200)):
                pltpu.sync_copy(acc_sp.at[pl.ds(base_r + o, nr)],
                                buf.at[pl.ds(0, nr)])
                pltpu.sync_copy(buf.at[pl.ds(0, nr)],
                                m_hbm.at[pl.ds(out_r + o, nr)])
                pltpu.sync_copy(den_sp.at[pl.ds(base_r + o, nr)],
                                exb.at[pl.ds(0, nr)])
                pltpu.sync_copy(exb.at[pl.ds(0, nr)],
                                den_hbm.at[pl.ds(out_r + o, nr)])

        plsc.subcore_barrier()


def _sca(h1flat, asp, adp, src, dst, srcoff4):
    mesh = plsc.VectorSubcoreMesh(core_axis_name="c", subcore_axis_name="s", num_cores=2, num_subcores=16)
    f = pl.kernel(
        _sca_body,
        out_type=[
            jax.ShapeDtypeStruct((4 * N, 64), F32),
            jax.ShapeDtypeStruct((4 * N, 8), F32),
        ],
        mesh=mesh,
        compiler_params=pltpu.CompilerParams(needs_layout_passes=False,
                                             use_tc_tiling_on_sc=False),
        scratch_types=[
            pltpu.VMEM((2, N), F32),
            pltpu.VMEM((2, N), F32),
            pltpu.VMEM((_KA,), I32),
            pltpu.VMEM((_KA,), I32),
            pltpu.VMEM((_KA,), I32),
            pltpu.VMEM((_KA, 64), F32),
            pltpu.VMEM((_KA, 8), F32),
            pltpu.VMEM((16,), F32),
            pltpu.VMEM_SHARED((N, 64), F32),
            pltpu.VMEM_SHARED((N, 8), F32),
            pltpu.SemaphoreType.DMA,
        ],
    )
    return f(h1flat, asp, adp, src, dst, srcoff4)


_BN2 = 400


def _tc2_body(m_ref, den_ref, b1_ref, w2_ref, att2_ref, h2_ref, a2_ref):
    b1 = b1_ref[...]
    feats = []
    for h in range(4):
        mh = m_ref[h]
        dh = den_ref[h, :, 0:1]
        denr = jnp.repeat(dh, 64, axis=1)
        o = jnp.where(denr > 0, mh / denr, 0.0) + b1[:, 64 * h:64 * (h + 1)]
        feats.append(jnp.where(o > 0, o, jnp.exp(o) - 1.0))
    hfeat = jnp.concatenate(feats, axis=1)
    h2_ref[...] = jnp.dot(hfeat, w2_ref[...], precision=lax.Precision.HIGHEST,
                          preferred_element_type=F32)
    a2w = jnp.dot(w2_ref[...], att2_ref[...], precision=lax.Precision.HIGHEST,
                  preferred_element_type=F32)
    a2_ref[...] = jnp.dot(hfeat, a2w, precision=lax.Precision.HIGHEST,
                          preferred_element_type=F32)


def _tc2(M, den, b1, W2, attcat2):
    nb = N // _BN2
    return pl.pallas_call(
        _tc2_body,
        grid=(nb,),
        in_specs=[
            pl.BlockSpec((4, _BN2, 64), lambda i: (0, i, 0)),
            pl.BlockSpec((4, _BN2, 8), lambda i: (0, i, 0)),
            pl.BlockSpec((1, 256), lambda i: (0, 0)),
            pl.BlockSpec((256, 64), lambda i: (0, 0)),
            pl.BlockSpec((64, 2), lambda i: (0, 0)),
        ],
        out_specs=[
            pl.BlockSpec((_BN2, 64), lambda i: (i, 0)),
            pl.BlockSpec((_BN2, 2), lambda i: (i, 0)),
        ],
        out_shape=[
            jax.ShapeDtypeStruct((N, 64), F32),
            jax.ShapeDtypeStruct((N, 2), F32),
        ],
    )(M, den, b1, W2, attcat2)


_KB = 1000
_EPT_B = E // 32
_NCH_B = _EPT_B // _KB


def _a2_max(a2_t, col, tmp16):
    def body(i, acc):
        rows = lax.iota(I32, 16) + 16 * i
        return jnp.maximum(acc, plsc.load_gather(a2_t, [rows, _full16(col)]))

    acc = lax.fori_loop(0, N // 16, body, jnp.full((16,), -jnp.inf, F32))
    return _bcast_max(acc, tmp16)


def _zero_flat(ref, n):
    z = jnp.zeros((16,), F32)

    def body(i, _):
        ref[pl.ds(16 * i, 16)] = z
        return 0

    lax.fori_loop(0, n // 16, body, 0)
    if n % 16:
        iota = lax.iota(I32, 16)
        plsc.store_scatter(ref, [iota + (n - 16)], z,
                           mask=iota >= (16 - n % 16))


def _scb_body(a2_hbm, src_hbm, dst_hbm, den2_hbm,
              a2_t, src_c, dst_c, exb, tmp16, den_sp):
    c = lax.axis_index("c")
    s = lax.axis_index("s")
    pltpu.sync_copy(a2_hbm, a2_t)
    c2 = _lrelu(_a2_max(a2_t, 0, tmp16) + _a2_max(a2_t, 1, tmp16))

    _zero_flat(exb, _KB)

    @pl.when(s < 10)
    def _():
        pltpu.sync_copy(exb, den_sp.at[pl.ds(s * 1000, 1000)])

    plsc.subcore_barrier()

    wid = s * 2 + c
    ebase = wid * _EPT_B

    def chunk(g, _):
        off = ebase + g * _KB
        pltpu.sync_copy(src_hbm.at[pl.ds(off, _KB)], src_c)
        pltpu.sync_copy(dst_hbm.at[pl.ds(off, _KB)], dst_c)

        def group(i, _):
            srcv = src_c[pl.ds(16 * i, 16)]
            dstv = dst_c[pl.ds(16 * i, 16)]
            av = plsc.load_gather(a2_t, [srcv, _full16(0)])
            dv = plsc.load_gather(a2_t, [dstv, _full16(1)])
            exb[pl.ds(16 * i, 16)] = jnp.exp(_lrelu(av + dv) - c2)
            return 0

        lax.fori_loop(0, _KB // 16, group, 0)
        iota = lax.iota(I32, 16)
        rows = iota + (_KB - 16)
        srcv = plsc.load_gather(src_c, [rows])
        dstv = plsc.load_gather(dst_c, [rows])
        av = plsc.load_gather(a2_t, [srcv, _full16(0)])
        dv = plsc.load_gather(a2_t, [dstv, _full16(1)])
        plsc.store_scatter(exb, [rows], jnp.exp(_lrelu(av + dv) - c2),
                           mask=iota >= 8)
        pltpu.sync_copy(exb, den_sp.at[dst_c], add=True)
        return 0

    lax.fori_loop(0, _NCH_B, chunk, 0)
    plsc.subcore_barrier()

    @pl.when(s < 10)
    def _():
        pltpu.sync_copy(den_sp.at[pl.ds(s * 1000, 1000)], exb)
        pltpu.sync_copy(exb, den2_hbm.at[pl.ds(c * N + s * 1000, 1000)])


def _scb(a2, src, dst):
    mesh = plsc.VectorSubcoreMesh(core_axis_name="c", subcore_axis_name="s", num_cores=2, num_subcores=16)
    f = pl.kernel(
        _scb_body,
        out_type=jax.ShapeDtypeStruct((2 * N,), F32),
        mesh=mesh,
        compiler_params=pltpu.CompilerParams(needs_layout_passes=False,
                                             use_tc_tiling_on_sc=False),
        scratch_types=[
            pltpu.VMEM((N, 2), F32),
            pltpu.VMEM((_KB,), I32),
            pltpu.VMEM((_KB,), I32),
            pltpu.VMEM((_KB,), F32),
            pltpu.VMEM((16,), F32),
            pltpu.VMEM_SHARED((N,), F32),
        ],
    )
    return f(a2, src, dst)


def _scc_body(a2_hbm, den2_hbm, src_hbm, dst_hbm, w_hbm,
              a2_t, den2_t, dpart, src_c, dst_c, cb, tmp16, w_sp):
    c = lax.axis_index("c")
    s = lax.axis_index("s")
    pltpu.sync_copy(a2_hbm, a2_t)
    c2 = _lrelu(_a2_max(a2_t, 0, tmp16) + _a2_max(a2_t, 1, tmp16))

    pltpu.sync_copy(den2_hbm.at[pl.ds(0, N)], den2_t)
    pltpu.sync_copy(den2_hbm.at[pl.ds(N, N)], dpart)

    def addp(i, _):
        sl = pl.ds(16 * i, 16)
        den2_t[sl] = den2_t[sl] + dpart[sl]
        return 0

    lax.fori_loop(0, N // 16, addp, 0)

    _zero_flat(cb, _KB)

    @pl.when(s < 10)
    def _():
        pltpu.sync_copy(cb, w_sp.at[pl.ds(s * 1000, 1000)])

    plsc.subcore_barrier()

    wid = s * 2 + c
    ebase = wid * _EPT_B

    def chunk(g, _):
        off = ebase + g * _KB
        pltpu.sync_copy(src_hbm.at[pl.ds(off, _KB)], src_c)
        pltpu.sync_copy(dst_hbm.at[pl.ds(off, _KB)], dst_c)

        def group(i, _):
            srcv = src_c[pl.ds(16 * i, 16)]
            dstv = dst_c[pl.ds(16 * i, 16)]
            av = plsc.load_gather(a2_t, [srcv, _full16(0)])
            dv = plsc.load_gather(a2_t, [dstv, _full16(1)])
            ex = jnp.exp(_lrelu(av + dv) - c2)
            den = plsc.load_gather(den2_t, [dstv])
            cb[pl.ds(16 * i, 16)] = ex / den
            return 0

        lax.fori_loop(0, _KB // 16, group, 0)
        iota = lax.iota(I32, 16)
        rows = iota + (_KB - 16)
        srcv = plsc.load_gather(src_c, [rows])
        dstv = plsc.load_gather(dst_c, [rows])
        av = plsc.load_gather(a2_t, [srcv, _full16(0)])
        dv = plsc.load_gather(a2_t, [dstv, _full16(1)])
        ex = jnp.exp(_lrelu(av + dv) - c2)
        den = plsc.load_gather(den2_t, [dstv])
        plsc.store_scatter(cb, [rows], ex / den, mask=iota >= 8)
        pltpu.sync_copy(cb, w_sp.at[src_c], add=True)
        return 0

    lax.fori_loop(0, _NCH_B, chunk, 0)
    plsc.subcore_barrier()

    @pl.when(s < 10)
    def _():
        pltpu.sync_copy(w_sp.at[pl.ds(s * 1000, 1000)], cb)
        pltpu.sync_copy(cb, w_hbm.at[pl.ds(c * N + s * 1000, 1000)])


def _scc(a2, den2, src, dst):
    mesh = plsc.VectorSubcoreMesh(core_axis_name="c", subcore_axis_name="s", num_cores=2, num_subcores=16)
    f = pl.kernel(
        _scc_body,
        out_type=jax.ShapeDtypeStruct((2 * N,), F32),
        mesh=mesh,
        compiler_params=pltpu.CompilerParams(needs_layout_passes=False,
                                             use_tc_tiling_on_sc=False),
        scratch_types=[
            pltpu.VMEM((N, 2), F32),
            pltpu.VMEM((N,), F32),
            pltpu.VMEM((N,), F32),
            pltpu.VMEM((_KB,), I32),
            pltpu.VMEM((_KB,), I32),
            pltpu.VMEM((_KB,), F32),
            pltpu.VMEM((16,), F32),
            pltpu.VMEM_SHARED((N,), F32),
        ],
    )
    return f(a2, den2, src, dst)


def _tc3_body(h2_ref, w_ref, b2_ref, sp_ref, wa_ref, ba_ref, wb_ref, bb_ref,
              wc_ref, bc_ref, o_ref):
    w = (w_ref[0:1, :] + w_ref[1:2, :])
    ge = jnp.dot(w, h2_ref[...], precision=lax.Precision.HIGHEST,
                 preferred_element_type=F32) / float(N) + b2_ref[...]
    cc = jnp.concatenate([ge, sp_ref[...]], axis=1)
    m = jnp.dot(cc, wa_ref[...], precision=lax.Precision.HIGHEST,
                preferred_element_type=F32) + ba_ref[...]
    m = jnp.maximum(m, 0.0)
    m = jnp.dot(m, wb_ref[...], precision=lax.Precision.HIGHEST,
                preferred_element_type=F32) + bb_ref[...]
    m = jnp.maximum(m, 0.0)
    o = jnp.dot(m, wc_ref[...], precision=lax.Precision.HIGHEST,
                preferred_element_type=F32) + bc_ref[...]
    o_ref[...] = 1.0 / (1.0 + jnp.exp(-o))


def _tc3(h2, wpart, b2, sp, Wa, ba, Wb, bb, Wc, bc):
    return pl.pallas_call(
        _tc3_body,
        out_shape=jax.ShapeDtypeStruct((1, 1), F32),
    )(h2, wpart, b2, sp, Wa, ba, Wb, bb, Wc, bc)


def kernel(x, edge_index, species, W1, att_src1, att_dst1, b1, W2, att_src2,
           att_dst2, b2, species_table, Wa, ba, Wb, bb, Wc, bc):
    src = edge_index[0]
    dst = edge_index[1]

    As1 = jnp.einsum("khc,hc->kh", W1.reshape(15, 4, 64), att_src1)
    Ad1 = jnp.einsum("khc,hc->kh", W1.reshape(15, 4, 64), att_dst1)
    AsT = As1.T.reshape(2, 2, 15)
    AdT = Ad1.T.reshape(2, 2, 15)

    W1h = W1.reshape(15, 4, 64).transpose(1, 0, 2)
    h1p, asp, adp = _tc1(x, x.T, W1h, AsT, AdT)

    srcoff4 = (src[None, :]
               + (jnp.arange(4, dtype=I32) * N)[:, None]).reshape(4 * E)
    Mflat, denflat = _sca(h1p.reshape(4 * N, 64), asp, adp, src, dst, srcoff4)
    M = Mflat.reshape(4, N, 64)
    den = denflat.reshape(4, N, 8)

    attcat2 = jnp.stack([att_src2[0], att_dst2[0]], axis=1)
    h2, a2 = _tc2(M, den, b1.reshape(1, 256), W2, attcat2)

    den2 = _scb(a2, src, dst)
    wflat = _scc(a2, den2, src, dst)

    sp = species_table[species][None, :]
    out = _tc3(h2, wflat.reshape(2, N), b2.reshape(1, 64), sp,
               Wa, ba.reshape(1, 128), Wb, bb.reshape(1, 64), Wc,
               bc.reshape(1, 1))
    return out.reshape(1)
